# Initial kernel scaffold; baseline (speedup 1.0000x reference)
import jax
import jax.numpy as jnp
from jax.experimental import pallas as pl


def kernel(x, edge_index, W1, b1, W2, b2, W3, b3, Wl, bl):
    raise NotImplementedError("write your pallas kernel here")


# speedup vs baseline: 13.4820x; 13.4820x over previous
import functools
import jax
import jax.numpy as jnp
from jax import lax
from jax.experimental import pallas as pl
from jax.experimental.pallas import tpu as pltpu
from jax.experimental.pallas import tpu_sc as plsc

N = 10000
E = 320000
NC, NS, LANES = 2, 16, 16
NW = NC * NS
EPW = E // NW


def _deg_body(dst_hbm, out_hbm, dst_v, acc_v):
    wid = lax.axis_index("s") * NC + lax.axis_index("c")
    pltpu.sync_copy(dst_hbm.at[pl.ds(wid * EPW, EPW)], dst_v)

    zeros = jnp.zeros((LANES,), jnp.float32)

    def zbody(i, _):
        acc_v[pl.ds(i * LANES, LANES)] = zeros
        return 0

    lax.fori_loop(0, N // LANES, zbody, 0)

    ones = jnp.ones((LANES,), jnp.float32)

    def ebody(i, _):
        d16 = dst_v[pl.ds(i * LANES, LANES)]
        plsc.addupdate_scatter(acc_v, [d16], ones)
        return 0

    lax.fori_loop(0, EPW // LANES, ebody, 0)
    pltpu.sync_copy(acc_v, out_hbm.at[wid])


@functools.lru_cache(maxsize=None)
def _deg_kernel():
    return functools.partial(
        pl.kernel,
        out_type=jax.ShapeDtypeStruct((NW, N), jnp.float32),
        mesh=plsc.VectorSubcoreMesh(core_axis_name="c", subcore_axis_name="s"),
        compiler_params=pltpu.CompilerParams(needs_layout_passes=False),
        scratch_types=[
            pltpu.VMEM((EPW,), jnp.int32),
            pltpu.VMEM((N,), jnp.float32),
        ],
    )(_deg_body)


def _make_agg(F):
    EBLK = 2000 if F >= 6 else EPW
    NBLK = EPW // EBLK

    def body(g_hbm, src_hbm, dst_hbm, out_hbm, g_v, acc_v, src_v, dst_v):
        wid = lax.axis_index("s") * NC + lax.axis_index("c")
        base = wid * EPW
        pltpu.sync_copy(g_hbm, g_v)

        zeros = jnp.zeros((LANES,), jnp.float32)

        def zbody(i, _):
            acc_v[pl.ds(i * LANES, LANES)] = zeros
            return 0

        lax.fori_loop(0, N * F // LANES, zbody, 0)

        def ebody(i, _):
            s16 = src_v[pl.ds(i * LANES, LANES)] * F
            d16 = dst_v[pl.ds(i * LANES, LANES)] * F
            for f in range(F):
                v = plsc.load_gather(g_v, [s16 + f])
                plsc.addupdate_scatter(acc_v, [d16 + f], v)
            return 0

        for b in range(NBLK):
            pltpu.sync_copy(src_hbm.at[pl.ds(base + b * EBLK, EBLK)], src_v)
            pltpu.sync_copy(dst_hbm.at[pl.ds(base + b * EBLK, EBLK)], dst_v)
            lax.fori_loop(0, EBLK // LANES, ebody, 0)

        pltpu.sync_copy(acc_v, out_hbm.at[wid])

    return functools.partial(
        pl.kernel,
        out_type=jax.ShapeDtypeStruct((NW, N * F), jnp.float32),
        mesh=plsc.VectorSubcoreMesh(core_axis_name="c", subcore_axis_name="s"),
        compiler_params=pltpu.CompilerParams(needs_layout_passes=False),
        scratch_types=[
            pltpu.VMEM((N * F,), jnp.float32),
            pltpu.VMEM((N * F,), jnp.float32),
            pltpu.VMEM((EBLK,), jnp.int32),
            pltpu.VMEM((EBLK,), jnp.int32),
        ],
    )(body)


_make_agg = functools.lru_cache(maxsize=None)(_make_agg)


def _mm_body(x_ref, w_ref, o_ref):
    o_ref[...] = jnp.dot(x_ref[...], w_ref[...],
                         preferred_element_type=jnp.float32)


def _tc_matmul(x, w):
    return pl.pallas_call(
        _mm_body,
        out_shape=jax.ShapeDtypeStruct((x.shape[0], w.shape[1]), jnp.float32),
    )(x, w)


def _dinv_body(degp_ref, o_ref):
    deg = jnp.sum(degp_ref[...], axis=0) + 1.0
    o_ref[...] = lax.rsqrt(deg)


def _tc_dinv(degp):
    return pl.pallas_call(
        _dinv_body,
        out_shape=jax.ShapeDtypeStruct((N,), jnp.float32),
    )(degp)


def _scale_body(dinv_ref, h_ref, o_ref):
    o_ref[...] = dinv_ref[...] * h_ref[...]


def _tc_scale(dinv2, h):
    return pl.pallas_call(
        _scale_body,
        out_shape=jax.ShapeDtypeStruct(h.shape, jnp.float32),
    )(dinv2, h)


NB = 400
NG = N // NB


def _layer1_body(sp_ref, dinv_ref, g_ref, b_ref, x_ref, gn_ref):
    s = jnp.sum(sp_ref[...], axis=0)
    u = dinv_ref[...] * (s + g_ref[...])
    xv = u + b_ref[...]
    x_ref[...] = xv
    gn_ref[...] = dinv_ref[...] * xv


def _tc_layer1(sp, dinv2, g, b):
    F = g.shape[1]
    return pl.pallas_call(
        _layer1_body,
        grid=(NG,),
        in_specs=[
            pl.BlockSpec((NW, NB, F), lambda i: (0, i, 0)),
            pl.BlockSpec((NB, 1), lambda i: (i, 0)),
            pl.BlockSpec((NB, F), lambda i: (i, 0)),
            pl.BlockSpec((1, F), lambda i: (0, 0)),
        ],
        out_specs=[
            pl.BlockSpec((NB, F), lambda i: (i, 0)),
            pl.BlockSpec((NB, F), lambda i: (i, 0)),
        ],
        out_shape=[
            jax.ShapeDtypeStruct((N, F), jnp.float32),
            jax.ShapeDtypeStruct((N, F), jnp.float32),
        ],
    )(sp, dinv2, g, b)


def _layer2_body(sp_ref, dinv_ref, g_ref, w_ref, b_ref, x_ref, gn_ref):
    s = jnp.sum(sp_ref[...], axis=0)
    u = dinv_ref[...] * (s + g_ref[...])
    xv = jnp.dot(u, w_ref[...], preferred_element_type=jnp.float32) + b_ref[...]
    x_ref[...] = xv
    gn_ref[...] = dinv_ref[...] * xv


def _tc_layer2(sp, dinv2, g, w, b):
    F = g.shape[1]
    FO = w.shape[1]
    return pl.pallas_call(
        _layer2_body,
        grid=(NG,),
        in_specs=[
            pl.BlockSpec((NW, NB, F), lambda i: (0, i, 0)),
            pl.BlockSpec((NB, 1), lambda i: (i, 0)),
            pl.BlockSpec((NB, F), lambda i: (i, 0)),
            pl.BlockSpec((F, FO), lambda i: (0, 0)),
            pl.BlockSpec((1, FO), lambda i: (0, 0)),
        ],
        out_specs=[
            pl.BlockSpec((NB, FO), lambda i: (i, 0)),
            pl.BlockSpec((NB, FO), lambda i: (i, 0)),
        ],
        out_shape=[
            jax.ShapeDtypeStruct((N, FO), jnp.float32),
            jax.ShapeDtypeStruct((N, FO), jnp.float32),
        ],
    )(sp, dinv2, g, w, b)


def _layer3_body(sp_ref, dinv_ref, g_ref, w_ref, b_ref, wl_ref, bl_ref,
                 x_ref, o_ref):
    s = jnp.sum(sp_ref[...], axis=0)
    u = dinv_ref[...] * (s + g_ref[...])
    xv = jnp.dot(u, w_ref[...], preferred_element_type=jnp.float32) + b_ref[...]
    x_ref[...] = xv
    o_ref[...] = jnp.dot(xv, wl_ref[...],
                         preferred_element_type=jnp.float32) + bl_ref[...]


def _tc_layer3(sp, dinv2, g, w, b, wl, bl):
    F = g.shape[1]
    FO = w.shape[1]
    FL = wl.shape[1]
    return pl.pallas_call(
        _layer3_body,
        grid=(NG,),
        in_specs=[
            pl.BlockSpec((NW, NB, F), lambda i: (0, i, 0)),
            pl.BlockSpec((NB, 1), lambda i: (i, 0)),
            pl.BlockSpec((NB, F), lambda i: (i, 0)),
            pl.BlockSpec((F, FO), lambda i: (0, 0)),
            pl.BlockSpec((1, FO), lambda i: (0, 0)),
            pl.BlockSpec((FO, FL), lambda i: (0, 0)),
            pl.BlockSpec((1, FL), lambda i: (0, 0)),
        ],
        out_specs=[
            pl.BlockSpec((NB, FO), lambda i: (i, 0)),
            pl.BlockSpec((NB, FL), lambda i: (i, 0)),
        ],
        out_shape=[
            jax.ShapeDtypeStruct((N, FO), jnp.float32),
            jax.ShapeDtypeStruct((N, FL), jnp.float32),
        ],
    )(sp, dinv2, g, w, b, wl, bl)


def kernel(x, edge_index, W1, b1, W2, b2, W3, b3, Wl, bl):
    src = edge_index[0].astype(jnp.int32)
    dst = edge_index[1].astype(jnp.int32)

    h1 = _tc_matmul(x, W1)
    degp = _deg_kernel()(dst)
    dinv = _tc_dinv(degp)
    dinv2 = dinv.reshape(N, 1)

    g1 = _tc_scale(dinv2, h1)
    agg5 = _make_agg(5)
    s1p = agg5(g1.reshape(-1), src, dst).reshape(NW, N, 5)
    x1, g2 = _tc_layer1(s1p, dinv2, g1, b1.reshape(1, 5))

    s2p = agg5(g2.reshape(-1), src, dst).reshape(NW, N, 5)
    x2, g3 = _tc_layer2(s2p, dinv2, g2, W2, b2.reshape(1, 6))

    s3p = _make_agg(6)(g3.reshape(-1), src, dst).reshape(NW, N, 6)
    x3, out = _tc_layer3(s3p, dinv2, g3, W3, b3.reshape(1, 7),
                         Wl, bl.reshape(1, 8))

    return (out, [x1, x2, x3])


# speedup vs baseline: 57.5530x; 4.2689x over previous
import functools
import jax
import jax.numpy as jnp
from jax import lax
from jax.experimental import pallas as pl
from jax.experimental.pallas import tpu as pltpu
from jax.experimental.pallas import tpu_sc as plsc

N = 10000
E = 320000
NC, NS, LANES = 2, 16, 16
NW = NC * NS
EPW = E // NW
ZUNROLL = 25
EUNROLL = 5


def _zero_vmem(ref, nwords):
    zeros = jnp.zeros((LANES,), jnp.float32)
    per_it = ZUNROLL * LANES
    assert nwords % per_it == 0

    def zbody(i, _):
        for j in range(ZUNROLL):
            ref[pl.ds(i * per_it + j * LANES, LANES)] = zeros
        return 0

    lax.fori_loop(0, nwords // per_it, zbody, 0)


def _deg_body(dst_hbm, out_hbm, dst_v, acc_v):
    wid = lax.axis_index("s") * NC + lax.axis_index("c")
    pltpu.sync_copy(dst_hbm.at[pl.ds(wid * EPW, EPW)], dst_v)
    _zero_vmem(acc_v, N)

    ones = jnp.ones((LANES,), jnp.float32)

    def ebody(i, _):
        for j in range(EUNROLL):
            d16 = dst_v[pl.ds((i * EUNROLL + j) * LANES, LANES)]
            plsc.addupdate_scatter(acc_v, [d16], ones)
        return 0

    lax.fori_loop(0, EPW // (LANES * EUNROLL), ebody, 0)
    pltpu.sync_copy(acc_v, out_hbm.at[wid])


@functools.lru_cache(maxsize=None)
def _deg_kernel():
    return functools.partial(
        pl.kernel,
        out_type=jax.ShapeDtypeStruct((NW, N), jnp.float32),
        mesh=plsc.VectorSubcoreMesh(core_axis_name="c", subcore_axis_name="s"),
        compiler_params=pltpu.CompilerParams(needs_layout_passes=False),
        scratch_types=[
            pltpu.VMEM((EPW,), jnp.int32),
            pltpu.VMEM((N,), jnp.float32),
        ],
    )(_deg_body)


def _make_agg(F):
    EBLK = 2000 if F >= 6 else EPW
    NBLK = EPW // EBLK

    def body(g_hbm, src_hbm, dst_hbm, out_hbm, g_v, acc_v, src_v, dst_v):
        wid = lax.axis_index("s") * NC + lax.axis_index("c")
        base = wid * EPW
        pltpu.sync_copy(g_hbm, g_v)
        _zero_vmem(acc_v, N * F)

        def ebody(i, _):
            for j in range(EUNROLL):
                off = (i * EUNROLL + j) * LANES
                s16 = src_v[pl.ds(off, LANES)]
                d16 = dst_v[pl.ds(off, LANES)]
                for f in range(F):
                    v = plsc.load_gather(g_v, [s16 + f * N])
                    plsc.addupdate_scatter(acc_v, [d16 + f * N], v)
            return 0

        for b in range(NBLK):
            pltpu.sync_copy(src_hbm.at[pl.ds(base + b * EBLK, EBLK)], src_v)
            pltpu.sync_copy(dst_hbm.at[pl.ds(base + b * EBLK, EBLK)], dst_v)
            lax.fori_loop(0, EBLK // (LANES * EUNROLL), ebody, 0)

        pltpu.sync_copy(acc_v, out_hbm.at[wid])

    return functools.partial(
        pl.kernel,
        out_type=jax.ShapeDtypeStruct((NW, N * F), jnp.float32),
        mesh=plsc.VectorSubcoreMesh(core_axis_name="c", subcore_axis_name="s"),
        compiler_params=pltpu.CompilerParams(needs_layout_passes=False),
        scratch_types=[
            pltpu.VMEM((N * F,), jnp.float32),
            pltpu.VMEM((N * F,), jnp.float32),
            pltpu.VMEM((EBLK,), jnp.int32),
            pltpu.VMEM((EBLK,), jnp.int32),
        ],
    )(body)


_make_agg = functools.lru_cache(maxsize=None)(_make_agg)


def _sum32(ref):
    s = ref[0]
    for k in range(1, NW):
        s = s + ref[k]
    return s


def _head_body(xt_ref, w1t_ref, degp_ref, dinv_ref, g1_ref):
    deg = _sum32(degp_ref) + 1.0
    dinv = lax.rsqrt(deg)
    dinv_ref[...] = dinv
    h1t = jnp.dot(w1t_ref[...], xt_ref[...],
                  preferred_element_type=jnp.float32)
    g1_ref[...] = dinv * h1t


def _tc_head(xt, w1t, degp):
    return pl.pallas_call(
        _head_body,
        out_shape=[
            jax.ShapeDtypeStruct((N,), jnp.float32),
            jax.ShapeDtypeStruct((5, N), jnp.float32),
        ],
    )(xt, w1t, degp)


def _layer1_body(sp_ref, dinv_ref, g_ref, b_ref, x_ref, gn_ref):
    dinv = dinv_ref[...]
    u = dinv * (_sum32(sp_ref) + g_ref[...])
    xv = u + b_ref[...]
    x_ref[...] = xv
    gn_ref[...] = dinv * xv


def _tc_layer1(sp, dinv, g, bt):
    F = g.shape[0]
    return pl.pallas_call(
        _layer1_body,
        out_shape=[
            jax.ShapeDtypeStruct((F, N), jnp.float32),
            jax.ShapeDtypeStruct((F, N), jnp.float32),
        ],
    )(sp, dinv, g, bt)


def _layer2_body(sp_ref, dinv_ref, g_ref, wt_ref, b_ref, x_ref, gn_ref):
    dinv = dinv_ref[...]
    u = dinv * (_sum32(sp_ref) + g_ref[...])
    xv = jnp.dot(wt_ref[...], u, preferred_element_type=jnp.float32) + b_ref[...]
    x_ref[...] = xv
    gn_ref[...] = dinv * xv


def _tc_layer2(sp, dinv, g, wt, bt):
    FO = wt.shape[0]
    return pl.pallas_call(
        _layer2_body,
        out_shape=[
            jax.ShapeDtypeStruct((FO, N), jnp.float32),
            jax.ShapeDtypeStruct((FO, N), jnp.float32),
        ],
    )(sp, dinv, g, wt, bt)


def _layer3_body(sp_ref, dinv_ref, g_ref, wt_ref, b_ref, wlt_ref, bl_ref,
                 x_ref, o_ref):
    dinv = dinv_ref[...]
    u = dinv * (_sum32(sp_ref) + g_ref[...])
    xv = jnp.dot(wt_ref[...], u, preferred_element_type=jnp.float32) + b_ref[...]
    x_ref[...] = xv
    o_ref[...] = jnp.dot(wlt_ref[...], xv,
                         preferred_element_type=jnp.float32) + bl_ref[...]


def _tc_layer3(sp, dinv, g, wt, bt, wlt, blt):
    FO = wt.shape[0]
    FL = wlt.shape[0]
    return pl.pallas_call(
        _layer3_body,
        out_shape=[
            jax.ShapeDtypeStruct((FO, N), jnp.float32),
            jax.ShapeDtypeStruct((FL, N), jnp.float32),
        ],
    )(sp, dinv, g, wt, bt, wlt, blt)


def kernel(x, edge_index, W1, b1, W2, b2, W3, b3, Wl, bl):
    src = edge_index[0].astype(jnp.int32)
    dst = edge_index[1].astype(jnp.int32)
    xt = x.T

    degp = _deg_kernel()(dst)
    dinv, g1 = _tc_head(xt, W1.T, degp)

    agg5 = _make_agg(5)
    s1p = agg5(g1.reshape(-1), src, dst).reshape(NW, 5, N)
    x1t, g2 = _tc_layer1(s1p, dinv, g1, b1.reshape(5, 1))

    s2p = agg5(g2.reshape(-1), src, dst).reshape(NW, 5, N)
    x2t, g3 = _tc_layer2(s2p, dinv, g2, W2.T, b2.reshape(6, 1))

    s3p = _make_agg(6)(g3.reshape(-1), src, dst).reshape(NW, 6, N)
    x3t, outt = _tc_layer3(s3p, dinv, g3, W3.T, b3.reshape(7, 1),
                           Wl.T, bl.reshape(8, 1))

    return (outt.T, [x1t.T, x2t.T, x3t.T])
